# baseline (speedup 1.0000x reference)
import functools

import jax
import jax.numpy as jnp
from jax import lax
from jax.experimental import pallas as pl
from jax.experimental.pallas import tpu as pltpu
from jax.experimental.pallas import tpu_sc as plsc

_WEIGHT = 1.0
_EPS = 1e-05
_LANES = 16


@functools.lru_cache(maxsize=None)
def _build(n_pred, n_tgt, n_joint, n_limb, n_match):
    n_workers = 32
    mpw = n_match // n_workers
    groups = mpw // _LANES
    assert n_joint == 17

    mesh = plsc.VectorSubcoreMesh(core_axis_name="c", subcore_axis_name="s")
    half_p = n_pred * n_joint // _LANES
    half_t = n_tgt * n_joint // _LANES

    @functools.partial(
        pl.kernel,
        out_type=jax.ShapeDtypeStruct((n_workers, 2 * _LANES), jnp.float32),
        mesh=mesh,
        compiler_params=pltpu.CompilerParams(needs_layout_passes=False,
                                             use_tc_tiling_on_sc=False),
        scratch_types=[
            pltpu.VMEM((mpw,), jnp.int32),
            pltpu.VMEM((mpw,), jnp.int32),
            pltpu.VMEM((4 * mpw,), jnp.int32),
            pltpu.VMEM((4 * mpw,), jnp.int32),
            pltpu.VMEM((2 * mpw,), jnp.int32),
            pltpu.VMEM((mpw,), jnp.int32),
            pltpu.VMEM((mpw,), jnp.int32),
            pltpu.VMEM((4 * mpw, _LANES), jnp.float32),
            pltpu.VMEM((4 * mpw, _LANES), jnp.float32),
            pltpu.VMEM((2 * mpw, _LANES), jnp.float32),
            pltpu.VMEM((mpw,), jnp.float32),
            pltpu.VMEM((mpw,), jnp.float32),
            pltpu.VMEM((n_limb * 2 * _LANES,), jnp.int32),
            pltpu.VMEM((2 * _LANES,), jnp.float32),
            pltpu.SemaphoreType.DMA,
            pltpu.SemaphoreType.DMA,
        ],
    )
    def limb_loss(pxy_hbm, txy_hbm, ta_hbm, vf_hbm, lc_hbm,
                  sid_hbm, tid_hbm, out_hbm,
                  idx_s, idx_t, gidx_s, gidx_t, gidx_f, off_s, off_t,
                  p_v, t_v, vf_v, ta_v, scale_v,
                  lc_v, part_v, sem_a, sem_b):
        wid = lax.axis_index("s") * 2 + lax.axis_index("c")
        base = wid * mpw

        pltpu.sync_copy(sid_hbm.at[pl.ds(base, mpw)], idx_s)
        pltpu.sync_copy(tid_hbm.at[pl.ds(base, mpw)], idx_t)
        cp_ta = pltpu.async_copy(ta_hbm.at[idx_t], ta_v, sem_a)
        pltpu.sync_copy(lc_hbm, lc_v)

        lane = lax.broadcasted_iota(jnp.int32, (_LANES,), 0)

        def build_body(g, _):
            m_vec = g * _LANES + lane
            m2 = 2 * m_vec
            ids_s = idx_s[pl.ds(g * _LANES, _LANES)]
            r0 = (17 * ids_s) >> 4
            plsc.store_scatter(gidx_s, [m2], r0)
            plsc.store_scatter(gidx_s, [m2 + 1], r0 + 1)
            plsc.store_scatter(gidx_s, [m2 + 2 * mpw], r0 + half_p)
            plsc.store_scatter(gidx_s, [m2 + 2 * mpw + 1], r0 + half_p + 1)
            off_s[pl.ds(g * _LANES, _LANES)] = ids_s & 15
            ids_t = idx_t[pl.ds(g * _LANES, _LANES)]
            r0t = (17 * ids_t) >> 4
            plsc.store_scatter(gidx_t, [m2], r0t)
            plsc.store_scatter(gidx_t, [m2 + 1], r0t + 1)
            plsc.store_scatter(gidx_t, [m2 + 2 * mpw], r0t + half_t)
            plsc.store_scatter(gidx_t, [m2 + 2 * mpw + 1], r0t + half_t + 1)
            plsc.store_scatter(gidx_f, [m2], r0t)
            plsc.store_scatter(gidx_f, [m2 + 1], r0t + 1)
            off_t[pl.ds(g * _LANES, _LANES)] = ids_t & 15
            return 0
        lax.fori_loop(0, groups, build_body, 0)

        copies = [
            pltpu.async_copy(pxy_hbm.at[gidx_s], p_v, sem_b),
            pltpu.async_copy(txy_hbm.at[gidx_t], t_v, sem_b),
            pltpu.async_copy(vf_hbm.at[gidx_f], vf_v, sem_b),
        ]

        cp_ta.wait()

        def scale_body(g, _):
            av = ta_v[pl.ds(g * _LANES, _LANES)]
            seed = 0x5F3759DF - (lax.bitcast_convert_type(av, jnp.int32) >> 1)
            y = lax.bitcast_convert_type(seed, jnp.float32)
            half = 0.5 * av
            y = y * (1.5 - half * y * y)
            y = y * (1.5 - half * y * y)
            y = y * (1.5 - half * y * y)
            scale_v[pl.ds(g * _LANES, _LANES)] = 1.0 / (av * y + _EPS)
            return 0
        lax.fori_loop(0, groups, scale_body, 0)

        for cp in copies:
            cp.wait()

        def group_body(g, carry):
            acc_n, acc_d = carry
            gb = g * _LANES
            m_vec = gb + lane
            m32 = m_vec << 5
            bfs = m32 + off_s[pl.ds(gb, _LANES)]
            bft = m32 + off_t[pl.ds(gb, _LANES)]
            sc = scale_v[pl.ds(gb, _LANES)]
            for l in range(n_limb):
                c_vs = lc_v[pl.ds((l * 2 + 0) * _LANES, _LANES)]
                c_vd = lc_v[pl.ds((l * 2 + 1) * _LANES, _LANES)]
                ts1 = bfs + c_vs
                ts2 = bfs + c_vd
                tt1 = bft + c_vs
                tt2 = bft + c_vd
                rs1, ls1 = ts1 >> 4, ts1 & 15
                rs2, ls2 = ts2 >> 4, ts2 & 15
                rt1, lt1 = tt1 >> 4, tt1 & 15
                rt2, lt2 = tt2 >> 4, tt2 & 15
                psx = plsc.load_gather(p_v, [rs1, ls1])
                psy = plsc.load_gather(p_v, [rs1 + 2 * mpw, ls1])
                pdx = plsc.load_gather(p_v, [rs2, ls2])
                pdy = plsc.load_gather(p_v, [rs2 + 2 * mpw, ls2])
                tsx = plsc.load_gather(t_v, [rt1, lt1])
                tsy = plsc.load_gather(t_v, [rt1 + 2 * mpw, lt1])
                tdx = plsc.load_gather(t_v, [rt2, lt2])
                tdy = plsc.load_gather(t_v, [rt2 + 2 * mpw, lt2])
                vfs = plsc.load_gather(vf_v, [rt1, lt1])
                vfd = plsc.load_gather(vf_v, [rt2, lt2])
                gx = (psx - pdx) - (tsx - tdx)
                gy = (psy - pdy) - (tsy - tdy)
                tl1 = jnp.abs(gx) + jnp.abs(gy)
                flag = vfs * vfd
                acc_n = acc_n + tl1 * flag * sc
                acc_d = acc_d + flag
            return acc_n, acc_d

        num, den = lax.fori_loop(
            0, groups, group_body,
            (jnp.zeros((_LANES,), jnp.float32),
             jnp.zeros((_LANES,), jnp.float32)))

        part_v[pl.ds(0, _LANES)] = num
        part_v[pl.ds(_LANES, _LANES)] = den
        pltpu.sync_copy(part_v, out_hbm.at[wid])

    return limb_loss


def kernel(pred_joints, target_joints, target_areas, visible_flags,
           limbs_table, src_ids, tgt_ids):
    n_pred, n_joint, _ = pred_joints.shape
    n_tgt = target_joints.shape[0]
    n_limb = limbs_table.shape[0]
    n_match = src_ids.shape[0]

    pj = pred_joints.astype(jnp.float32)
    tj = target_joints.astype(jnp.float32)
    pxy = pj.reshape(-1, 2).T.reshape(-1, _LANES)
    txy = tj.reshape(-1, 2).T.reshape(-1, _LANES)
    vfg = visible_flags.astype(jnp.float32).reshape(-1, _LANES)

    sj = limbs_table[:, 0].astype(jnp.int32)
    dj = limbs_table[:, 1].astype(jnp.int32)
    lc = jnp.stack([sj, dj], axis=1)
    lc = jnp.broadcast_to(lc[:, :, None], (n_limb, 2, _LANES))
    lc = lc.reshape(n_limb * 2 * _LANES)

    fn = _build(n_pred, n_tgt, n_joint, n_limb, n_match)
    part = fn(pxy, txy, target_areas.astype(jnp.float32), vfg, lc,
              src_ids.astype(jnp.int32), tgt_ids.astype(jnp.int32))
    num = part[:, :_LANES].sum()
    den = part[:, _LANES:].sum()
    return (num / (den + _EPS)) * _WEIGHT


# speedup vs baseline: 1.1230x; 1.1230x over previous
import functools

import jax
import jax.numpy as jnp
from jax import lax
from jax.experimental import pallas as pl
from jax.experimental.pallas import tpu as pltpu
from jax.experimental.pallas import tpu_sc as plsc

_WEIGHT = 1.0
_EPS = 1e-05
_LANES = 16


@functools.lru_cache(maxsize=None)
def _build(n_pred, n_tgt, n_joint, n_limb, n_match):
    n_workers = 32
    mpw = n_match // n_workers
    groups = mpw // _LANES
    assert n_joint == 17

    mesh = plsc.VectorSubcoreMesh(core_axis_name="c", subcore_axis_name="s")

    @functools.partial(
        pl.kernel,
        out_type=jax.ShapeDtypeStruct((n_workers, 2 * _LANES), jnp.float32),
        mesh=mesh,
        compiler_params=pltpu.CompilerParams(needs_layout_passes=False,
                                             use_tc_tiling_on_sc=False),
        scratch_types=[
            pltpu.VMEM((mpw,), jnp.int32),
            pltpu.VMEM((mpw,), jnp.int32),
            pltpu.VMEM((2 * mpw,), jnp.int32),
            pltpu.VMEM((2 * mpw,), jnp.int32),
            pltpu.VMEM((mpw,), jnp.int32),
            pltpu.VMEM((mpw,), jnp.int32),
            pltpu.VMEM((2 * mpw, _LANES), jnp.float32),
            pltpu.VMEM((2 * mpw, _LANES), jnp.float32),
            pltpu.VMEM((2 * mpw, _LANES), jnp.float32),
            pltpu.VMEM((2 * mpw, _LANES), jnp.float32),
            pltpu.VMEM((2 * mpw, _LANES), jnp.float32),
            pltpu.VMEM((mpw,), jnp.float32),
            pltpu.VMEM((mpw,), jnp.float32),
            pltpu.VMEM((n_limb * 2 * _LANES,), jnp.int32),
            pltpu.VMEM((2 * _LANES,), jnp.float32),
            pltpu.SemaphoreType.DMA,
            pltpu.SemaphoreType.DMA,
        ],
    )
    def limb_loss(px_hbm, py_hbm, tx_hbm, ty_hbm, ta_hbm, vf_hbm, lc_hbm,
                  sid_hbm, tid_hbm, out_hbm,
                  idx_s, idx_t, gidx_s, gidx_t, off_s, off_t,
                  px_v, py_v, tx_v, ty_v, vf_v, ta_v, scale_v,
                  lc_v, part_v, sem_a, sem_b):
        wid = lax.axis_index("s") * 2 + lax.axis_index("c")
        base = wid * mpw

        pltpu.sync_copy(sid_hbm.at[pl.ds(base, mpw)], idx_s)
        pltpu.sync_copy(tid_hbm.at[pl.ds(base, mpw)], idx_t)
        cp_ta = pltpu.async_copy(ta_hbm.at[idx_t], ta_v, sem_a)
        pltpu.sync_copy(lc_hbm, lc_v)

        lane = lax.broadcasted_iota(jnp.int32, (_LANES,), 0)

        def build_body(g, _):
            m_vec = g * _LANES + lane
            m2 = 2 * m_vec
            ids_s = idx_s[pl.ds(g * _LANES, _LANES)]
            r0 = (17 * ids_s) >> 4
            plsc.store_scatter(gidx_s, [m2], r0)
            plsc.store_scatter(gidx_s, [m2 + 1], r0 + 1)
            off_s[pl.ds(g * _LANES, _LANES)] = ids_s & 15
            ids_t = idx_t[pl.ds(g * _LANES, _LANES)]
            r0t = (17 * ids_t) >> 4
            plsc.store_scatter(gidx_t, [m2], r0t)
            plsc.store_scatter(gidx_t, [m2 + 1], r0t + 1)
            off_t[pl.ds(g * _LANES, _LANES)] = ids_t & 15
            return 0
        lax.fori_loop(0, groups, build_body, 0)

        copies = [
            pltpu.async_copy(px_hbm.at[gidx_s], px_v, sem_b),
            pltpu.async_copy(py_hbm.at[gidx_s], py_v, sem_b),
            pltpu.async_copy(tx_hbm.at[gidx_t], tx_v, sem_b),
            pltpu.async_copy(ty_hbm.at[gidx_t], ty_v, sem_b),
            pltpu.async_copy(vf_hbm.at[gidx_t], vf_v, sem_b),
        ]

        cp_ta.wait()

        def scale_body(g, _):
            av = ta_v[pl.ds(g * _LANES, _LANES)]
            seed = 0x5F3759DF - (lax.bitcast_convert_type(av, jnp.int32) >> 1)
            y = lax.bitcast_convert_type(seed, jnp.float32)
            half = 0.5 * av
            y = y * (1.5 - half * y * y)
            y = y * (1.5 - half * y * y)
            y = y * (1.5 - half * y * y)
            scale_v[pl.ds(g * _LANES, _LANES)] = 1.0 / (av * y + _EPS)
            return 0
        lax.fori_loop(0, groups, scale_body, 0)

        for cp in copies:
            cp.wait()

        def group_body(g, carry):
            acc_n, acc_d = carry
            gb = g * _LANES
            m_vec = gb + lane
            m32 = m_vec << 5
            bfs = m32 + off_s[pl.ds(gb, _LANES)]
            bft = m32 + off_t[pl.ds(gb, _LANES)]
            sc = scale_v[pl.ds(gb, _LANES)]
            for l in range(n_limb):
                c_vs = lc_v[pl.ds((l * 2 + 0) * _LANES, _LANES)]
                c_vd = lc_v[pl.ds((l * 2 + 1) * _LANES, _LANES)]
                ts1 = bfs + c_vs
                ts2 = bfs + c_vd
                tt1 = bft + c_vs
                tt2 = bft + c_vd
                rs1, ls1 = ts1 >> 4, ts1 & 15
                rs2, ls2 = ts2 >> 4, ts2 & 15
                rt1, lt1 = tt1 >> 4, tt1 & 15
                rt2, lt2 = tt2 >> 4, tt2 & 15
                psx = plsc.load_gather(px_v, [rs1, ls1])
                psy = plsc.load_gather(py_v, [rs1, ls1])
                pdx = plsc.load_gather(px_v, [rs2, ls2])
                pdy = plsc.load_gather(py_v, [rs2, ls2])
                tsx = plsc.load_gather(tx_v, [rt1, lt1])
                tsy = plsc.load_gather(ty_v, [rt1, lt1])
                tdx = plsc.load_gather(tx_v, [rt2, lt2])
                tdy = plsc.load_gather(ty_v, [rt2, lt2])
                vfs = plsc.load_gather(vf_v, [rt1, lt1])
                vfd = plsc.load_gather(vf_v, [rt2, lt2])
                gx = (psx - pdx) - (tsx - tdx)
                gy = (psy - pdy) - (tsy - tdy)
                tl1 = jnp.abs(gx) + jnp.abs(gy)
                flag = vfs * vfd
                acc_n = acc_n + tl1 * flag * sc
                acc_d = acc_d + flag
            return acc_n, acc_d

        num, den = lax.fori_loop(
            0, groups, group_body,
            (jnp.zeros((_LANES,), jnp.float32),
             jnp.zeros((_LANES,), jnp.float32)))

        part_v[pl.ds(0, _LANES)] = num
        part_v[pl.ds(_LANES, _LANES)] = den
        pltpu.sync_copy(part_v, out_hbm.at[wid])

    return limb_loss


def kernel(pred_joints, target_joints, target_areas, visible_flags,
           limbs_table, src_ids, tgt_ids):
    n_pred, n_joint, _ = pred_joints.shape
    n_tgt = target_joints.shape[0]
    n_limb = limbs_table.shape[0]
    n_match = src_ids.shape[0]

    pj = pred_joints.astype(jnp.float32)
    tj = target_joints.astype(jnp.float32)
    px = pj[:, :, 0].reshape(-1, _LANES)
    py = pj[:, :, 1].reshape(-1, _LANES)
    tx = tj[:, :, 0].reshape(-1, _LANES)
    ty = tj[:, :, 1].reshape(-1, _LANES)
    vfg = visible_flags.astype(jnp.float32).reshape(-1, _LANES)

    sj = limbs_table[:, 0].astype(jnp.int32)
    dj = limbs_table[:, 1].astype(jnp.int32)
    lc = jnp.stack([sj, dj], axis=1)
    lc = jnp.broadcast_to(lc[:, :, None], (n_limb, 2, _LANES))
    lc = lc.reshape(n_limb * 2 * _LANES)

    fn = _build(n_pred, n_tgt, n_joint, n_limb, n_match)
    part = fn(px, py, tx, ty, target_areas.astype(jnp.float32), vfg, lc,
              src_ids.astype(jnp.int32), tgt_ids.astype(jnp.int32))
    num = part[:, :_LANES].sum()
    den = part[:, _LANES:].sum()
    return (num / (den + _EPS)) * _WEIGHT
